# Initial kernel scaffold; baseline (speedup 1.0000x reference)
import jax
import jax.numpy as jnp
from jax.experimental import pallas as pl


def kernel(indices, hashed_weight):
    raise NotImplementedError("write your pallas kernel here")


# speedup vs baseline: 179.9751x; 179.9751x over previous
import functools

import numpy as np
import jax
import jax.numpy as jnp
from jax import lax
from jax.experimental import pallas as pl
from jax.experimental.pallas import tpu as pltpu
from jax.experimental.pallas import tpu_sc as plsc

_P = 2038074743
_r = np.random.RandomState(1024)
_rn = np.concatenate([np.array([2038074743]), _r.randint(0, 2038074743, (50,))])
_A, _B = int(_rn[1]), int(_rn[2])
_D = 64
_W = int(1000000 * _D * (1.0 / _D) + 1)
_N = 4096 * 50

_T1 = np.array([(_A * _D * 1024 * h + _B) % _P for h in range(1024)], dtype=np.int32)
_T2 = np.array([(_A * _D * l) % _P for l in range(1024)], dtype=np.int32)
_OFF = [int((_A * d) % _P) for d in range(_D)]
_RECIP = np.float32((1.0 / _W) * (1.0 - 2.0 ** -20))
_PU = np.uint32(_P)

_NC, _NS, _L = 2, 16, 16
_NW = _NC * _NS
_ROWS_PER_TILE = _N // _NW
_C = 256
_NCHUNK = _ROWS_PER_TILE // _C

_mesh = plsc.VectorSubcoreMesh(core_axis_name="c", subcore_axis_name="s")


@functools.partial(
    pl.kernel,
    out_type=jax.ShapeDtypeStruct((_N * _D,), jnp.float32),
    mesh=_mesh,
    compiler_params=pltpu.CompilerParams(needs_layout_passes=False),
    scratch_types=[
        pltpu.VMEM((1024,), jnp.int32),
        pltpu.VMEM((1024,), jnp.int32),
        pltpu.VMEM((_C,), jnp.int32),
        pltpu.VMEM((_C,), jnp.int32),
        pltpu.VMEM((_C * _D,), jnp.int32),
        pltpu.VMEM((_C * _D,), jnp.float32),
        pltpu.SemaphoreType.DMA,
    ],
)
def _emb_kernel(idx_hbm, w_hbm, t1_hbm, t2_hbm, out_hbm,
                t1_v, t2_v, idx_v, b_v, slot_v, val_v, sem):
    wid = lax.axis_index("s") * _NC + lax.axis_index("c")
    base_row = wid * np.int32(_ROWS_PER_TILE)
    pltpu.sync_copy(t1_hbm, t1_v)
    pltpu.sync_copy(t2_hbm, t2_v)

    lane64 = lax.iota(jnp.int32, 16) * np.int32(_D)

    def chunk_body(k, carry):
        row0 = base_row + k * np.int32(_C)
        pltpu.sync_copy(idx_hbm.at[pl.ds(row0, _C)], idx_v)

        def b_body(i, c):
            v = idx_v[pl.ds(i * np.int32(16), 16)]
            hi = lax.shift_right_logical(v, np.int32(10))
            lo = lax.bitwise_and(v, np.int32(1023))
            t1 = plsc.load_gather(t1_v, [hi])
            t2 = plsc.load_gather(t2_v, [lo])
            s = lax.bitcast_convert_type(t1 + t2, jnp.uint32)
            b = jnp.where(s >= _PU, s - _PU, s)
            b_v[pl.ds(i * np.int32(16), 16)] = lax.bitcast_convert_type(b, jnp.int32)
            return c

        lax.fori_loop(jnp.int32(0), jnp.int32(_C // 16), b_body, jnp.int32(0))

        def rb_body(rb, c):
            bvec = lax.bitcast_convert_type(b_v[pl.ds(rb * np.int32(16), 16)], jnp.uint32)
            pos0 = lane64 + rb * np.int32(16 * _D)
            for d in range(_D):
                h0 = bvec + np.uint32(_OFF[d])
                h = jnp.where(h0 >= _PU, h0 - _PU, h0)
                hi32 = lax.bitcast_convert_type(h, jnp.int32)
                q = (hi32.astype(jnp.float32) * _RECIP).astype(jnp.int32)
                r = hi32 - q * np.int32(_W)
                slot = jnp.where(r >= np.int32(_W), r - np.int32(_W), r)
                plsc.store_scatter(slot_v, [pos0 + np.int32(d)], slot)
            return c

        lax.fori_loop(jnp.int32(0), jnp.int32(_C // 16), rb_body, jnp.int32(0))

        pltpu.async_copy(w_hbm.at[slot_v], val_v, sem).wait()
        pltpu.sync_copy(val_v, out_hbm.at[pl.ds(row0 * np.int32(_D), _C * _D)])
        return carry

    lax.fori_loop(jnp.int32(0), jnp.int32(_NCHUNK), chunk_body, jnp.int32(0))


def kernel(indices, hashed_weight):
    idx32 = indices.reshape(-1).astype(jnp.int32)
    w = hashed_weight.astype(jnp.float32)
    out = _emb_kernel(idx32, w, jnp.asarray(_T1), jnp.asarray(_T2))
    return out.reshape(_N, _D)


# speedup vs baseline: 207.4839x; 1.1528x over previous
import functools

import numpy as np
import jax
import jax.numpy as jnp
from jax import lax
from jax.experimental import pallas as pl
from jax.experimental.pallas import tpu as pltpu
from jax.experimental.pallas import tpu_sc as plsc

_P = 2038074743
_r = np.random.RandomState(1024)
_rn = np.concatenate([np.array([2038074743]), _r.randint(0, 2038074743, (50,))])
_A, _B = int(_rn[1]), int(_rn[2])
_D = 64
_W = int(1000000 * _D * (1.0 / _D) + 1)
_N = 4096 * 50

_T1 = np.array([(_A * _D * 1024 * h + _B) % _P for h in range(1024)], dtype=np.int32)
_T2 = np.array([(_A * _D * l) % _P for l in range(1024)], dtype=np.int32)
_OFF = [int((_A * d) % _P) for d in range(_D)]
_RECIP = np.float32((1.0 / _W) * (1.0 - 2.0 ** -20))
_PU = np.uint32(_P)

_NC, _NS = 2, 16
_NW = _NC * _NS
_ROWS_PER_TILE = _N // _NW
_C = 256
_CD = _C * _D
_NCHUNK = _ROWS_PER_TILE // _C

_mesh = plsc.VectorSubcoreMesh(core_axis_name="c", subcore_axis_name="s")


@functools.partial(
    pl.kernel,
    out_type=jax.ShapeDtypeStruct((_N * _D,), jnp.float32),
    mesh=_mesh,
    compiler_params=pltpu.CompilerParams(needs_layout_passes=False),
    scratch_types=[
        pltpu.VMEM((1024,), jnp.int32),
        pltpu.VMEM((1024,), jnp.int32),
        pltpu.VMEM((2 * _C,), jnp.int32),
        pltpu.VMEM((_C,), jnp.int32),
        pltpu.VMEM((_CD,), jnp.int32),
        pltpu.VMEM((_CD,), jnp.int32),
        pltpu.VMEM((_CD,), jnp.float32),
        pltpu.VMEM((_CD,), jnp.float32),
        pltpu.SemaphoreType.DMA,
        pltpu.SemaphoreType.DMA,
        pltpu.SemaphoreType.DMA,
        pltpu.SemaphoreType.DMA,
    ],
)
def _emb_kernel(idx2_hbm, w_hbm, t1_hbm, t2_hbm, out_hbm,
                t1_v, t2_v, idx2_v, b_v, slot0_v, slot1_v, val0_v, val1_v,
                sem_g0, sem_g1, sem_o0, sem_o1):
    wid = lax.axis_index("s") * _NC + lax.axis_index("c")
    base_row = wid * np.int32(_ROWS_PER_TILE)
    pltpu.sync_copy(t1_hbm, t1_v)
    pltpu.sync_copy(t2_hbm, t2_v)

    lane64 = lax.iota(jnp.int32, 16) * np.int32(_D)
    lane2 = lax.iota(jnp.int32, 16) * np.int32(2)
    def compute_chunk(k, slot_v):
        row0 = base_row + k * np.int32(_C)
        pltpu.sync_copy(idx2_hbm.at[pl.ds(row0 * np.int32(2), 2 * _C)], idx2_v)

        def b_body(i, c):
            v = plsc.load_gather(idx2_v, [lane2 + i * np.int32(32)])
            hi = lax.shift_right_logical(v, np.int32(10))
            lo = lax.bitwise_and(v, np.int32(1023))
            t1 = plsc.load_gather(t1_v, [hi])
            t2 = plsc.load_gather(t2_v, [lo])
            s = lax.bitcast_convert_type(t1 + t2, jnp.uint32)
            b = jnp.where(s >= _PU, s - _PU, s)
            b_v[pl.ds(i * np.int32(16), 16)] = lax.bitcast_convert_type(b, jnp.int32)
            return c

        lax.fori_loop(jnp.int32(0), jnp.int32(_C // 16), b_body, jnp.int32(0))

        def rb_body(rb, c):
            bvec = lax.bitcast_convert_type(b_v[pl.ds(rb * np.int32(16), 16)], jnp.uint32)
            pos0 = lane64 + rb * np.int32(16 * _D)
            for d in range(_D):
                h0 = bvec + np.uint32(_OFF[d])
                h = jnp.where(h0 >= _PU, h0 - _PU, h0)
                hi32 = lax.bitcast_convert_type(h, jnp.int32)
                q = (hi32.astype(jnp.float32) * _RECIP).astype(jnp.int32)
                r = hi32 - q * np.int32(_W)
                slot = jnp.where(r >= np.int32(_W), r - np.int32(_W), r)
                plsc.store_scatter(slot_v, [pos0 + np.int32(d)], slot)
            return c

        lax.fori_loop(jnp.int32(0), jnp.int32(_C // 16), rb_body, jnp.int32(0))

    def gather_start(slot_v, val_v, sem):
        return pltpu.async_copy(w_hbm.at[slot_v], val_v, sem)

    def gather_wait(slot_v, val_v, sem):
        pltpu.make_async_copy(w_hbm.at[slot_v], val_v, sem).wait()

    def out_copy_start(k, val_v, sem):
        row0 = base_row + k * np.int32(_C)
        return pltpu.async_copy(
            val_v, out_hbm.at[pl.ds(row0 * np.int32(_D), _CD)], sem)

    def out_copy_wait(k, val_v, sem):
        row0 = base_row + k * np.int32(_C)
        pltpu.make_async_copy(
            val_v, out_hbm.at[pl.ds(row0 * np.int32(_D), _CD)], sem).wait()

    compute_chunk(jnp.int32(0), slot0_v)
    gather_start(slot0_v, val0_v, sem_g0)

    def sblock(s, carry):
        k1 = np.int32(2) * s + np.int32(1)
        k2 = k1 + np.int32(1)
        compute_chunk(k1, slot1_v)
        gather_wait(slot0_v, val0_v, sem_g0)
        out_copy_start(k1 - np.int32(1), val0_v, sem_o0)

        @pl.when(s >= np.int32(1))
        def _():
            out_copy_wait(k1 - np.int32(2), val1_v, sem_o1)

        gather_start(slot1_v, val1_v, sem_g1)
        compute_chunk(k2, slot0_v)
        gather_wait(slot1_v, val1_v, sem_g1)
        out_copy_start(k1, val1_v, sem_o1)
        out_copy_wait(k2 - np.int32(2), val0_v, sem_o0)
        gather_start(slot0_v, val0_v, sem_g0)
        return carry

    nsb = (_NCHUNK - 1) // 2
    lax.fori_loop(jnp.int32(0), jnp.int32(nsb), sblock, jnp.int32(0))

    last = np.int32(_NCHUNK - 1)
    gather_wait(slot0_v, val0_v, sem_g0)
    out_copy_start(last, val0_v, sem_o0)
    out_copy_wait(last - np.int32(1), val1_v, sem_o1)
    out_copy_wait(last, val0_v, sem_o0)


def kernel(indices, hashed_weight):
    idx2 = lax.bitcast_convert_type(indices.reshape(-1), jnp.int32).reshape(-1)
    w = hashed_weight.astype(jnp.float32)
    out = _emb_kernel(idx2, w, jnp.asarray(_T1), jnp.asarray(_T2))
    return out.reshape(_N, _D)


# speedup vs baseline: 379.6783x; 1.8299x over previous
import functools

import numpy as np
import jax
import jax.numpy as jnp
from jax import lax
from jax.experimental import pallas as pl
from jax.experimental.pallas import tpu as pltpu
from jax.experimental.pallas import tpu_sc as plsc

_P = 2038074743
_r = np.random.RandomState(1024)
_rn = np.concatenate([np.array([2038074743]), _r.randint(0, 2038074743, (50,))])
_A, _B = int(_rn[1]), int(_rn[2])
_D = 64
_W = int(1000000 * _D * (1.0 / _D) + 1)
_N = 4096 * 50

_T1 = np.array([(_A * _D * 1024 * h + _B) % _P for h in range(1024)], dtype=np.int32)
_T2 = np.array([(_A * _D * l) % _P for l in range(1024)], dtype=np.int32)
_OFF = [int((_A * d) % _P) for d in range(_D)]
_RECIP = np.float32((1.0 / _W) * (1.0 - 2.0 ** -20))
_PU = np.uint32(_P)

_NC, _NS = 2, 16
_NW = _NC * _NS
_ROWS_PER_TILE = _N // _NW
_C = 128
_CD = _C * _D
_NCHUNK = _ROWS_PER_TILE // _C

_mesh = plsc.VectorSubcoreMesh(core_axis_name="c", subcore_axis_name="s")


@functools.partial(
    pl.kernel,
    out_type=jax.ShapeDtypeStruct((_N * _D,), jnp.float32),
    mesh=_mesh,
    compiler_params=pltpu.CompilerParams(needs_layout_passes=False),
    scratch_types=[
        pltpu.VMEM((1024,), jnp.int32),
        pltpu.VMEM((1024,), jnp.int32),
        pltpu.VMEM((_C,), jnp.int32),
        pltpu.VMEM((_C,), jnp.int32),
        pltpu.VMEM((_CD,), jnp.int32),
        pltpu.VMEM((_CD,), jnp.int32),
        pltpu.VMEM((_CD,), jnp.float32),
        pltpu.VMEM((_CD,), jnp.float32),
        pltpu.VMEM_SHARED((_W,), jnp.float32),
        pltpu.SemaphoreType.DMA,
        pltpu.SemaphoreType.DMA,
        pltpu.SemaphoreType.DMA,
        pltpu.SemaphoreType.DMA,
    ],
)
def _emb_kernel(idx_hbm, w_hbm, t1_hbm, t2_hbm, out_hbm,
                t1_v, t2_v, idx_v, b_v, slot0_v, slot1_v, val0_v, val1_v,
                w_sp, sem_g0, sem_g1, sem_o0, sem_o1):
    wid = lax.axis_index("s") * _NC + lax.axis_index("c")
    base_row = wid * np.int32(_ROWS_PER_TILE)
    pltpu.sync_copy(t1_hbm, t1_v)
    pltpu.sync_copy(t2_hbm, t2_v)

    @pl.when(lax.axis_index("s") == jnp.int32(0))
    def _():
        pltpu.sync_copy(w_hbm, w_sp)
    plsc.subcore_barrier()

    lane64 = lax.iota(jnp.int32, 16) * np.int32(_D)
    def compute_chunk(k, slot_v):
        row0 = base_row + k * np.int32(_C)
        pltpu.sync_copy(idx_hbm.at[pl.ds(row0, _C)], idx_v)

        def b_body(i, c):
            v = idx_v[pl.ds(i * np.int32(16), 16)]
            hi = lax.shift_right_logical(v, np.int32(10))
            lo = lax.bitwise_and(v, np.int32(1023))
            t1 = plsc.load_gather(t1_v, [hi])
            t2 = plsc.load_gather(t2_v, [lo])
            s = lax.bitcast_convert_type(t1 + t2, jnp.uint32)
            b = jnp.where(s >= _PU, s - _PU, s)
            b_v[pl.ds(i * np.int32(16), 16)] = lax.bitcast_convert_type(b, jnp.int32)
            return c

        lax.fori_loop(jnp.int32(0), jnp.int32(_C // 16), b_body, jnp.int32(0))

        def rb_body(rb, c):
            bvec = lax.bitcast_convert_type(b_v[pl.ds(rb * np.int32(16), 16)], jnp.uint32)
            pos0 = lane64 + rb * np.int32(16 * _D)
            for d in range(_D):
                h0 = bvec + np.uint32(_OFF[d])
                h = jnp.where(h0 >= _PU, h0 - _PU, h0)
                hi32 = lax.bitcast_convert_type(h, jnp.int32)
                q = (hi32.astype(jnp.float32) * _RECIP).astype(jnp.int32)
                r = hi32 - q * np.int32(_W)
                slot = jnp.where(r >= np.int32(_W), r - np.int32(_W), r)
                plsc.store_scatter(slot_v, [pos0 + np.int32(d)], slot)
            return c

        lax.fori_loop(jnp.int32(0), jnp.int32(_C // 16), rb_body, jnp.int32(0))

    def gather_start(slot_v, val_v, sem):
        return pltpu.async_copy(w_sp.at[slot_v], val_v, sem)

    def gather_wait(slot_v, val_v, sem):
        pltpu.make_async_copy(w_sp.at[slot_v], val_v, sem).wait()

    def out_copy_start(k, val_v, sem):
        row0 = base_row + k * np.int32(_C)
        return pltpu.async_copy(
            val_v, out_hbm.at[pl.ds(row0 * np.int32(_D), _CD)], sem)

    def out_copy_wait(k, val_v, sem):
        row0 = base_row + k * np.int32(_C)
        pltpu.make_async_copy(
            val_v, out_hbm.at[pl.ds(row0 * np.int32(_D), _CD)], sem).wait()

    compute_chunk(jnp.int32(0), slot0_v)
    gather_start(slot0_v, val0_v, sem_g0)

    def sblock(s, carry):
        k1 = np.int32(2) * s + np.int32(1)
        k2 = k1 + np.int32(1)
        compute_chunk(k1, slot1_v)
        gather_wait(slot0_v, val0_v, sem_g0)
        out_copy_start(k1 - np.int32(1), val0_v, sem_o0)

        @pl.when(s >= np.int32(1))
        def _():
            out_copy_wait(k1 - np.int32(2), val1_v, sem_o1)

        gather_start(slot1_v, val1_v, sem_g1)
        compute_chunk(k2, slot0_v)
        gather_wait(slot1_v, val1_v, sem_g1)
        out_copy_start(k1, val1_v, sem_o1)
        out_copy_wait(k2 - np.int32(2), val0_v, sem_o0)
        gather_start(slot0_v, val0_v, sem_g0)
        return carry

    nsb = (_NCHUNK - 2) // 2
    lax.fori_loop(jnp.int32(0), jnp.int32(nsb), sblock, jnp.int32(0))

    last = np.int32(_NCHUNK - 1)
    compute_chunk(last, slot1_v)
    gather_wait(slot0_v, val0_v, sem_g0)
    out_copy_start(last - np.int32(1), val0_v, sem_o0)
    out_copy_wait(last - np.int32(2), val1_v, sem_o1)
    gather_start(slot1_v, val1_v, sem_g1)
    gather_wait(slot1_v, val1_v, sem_g1)
    out_copy_start(last, val1_v, sem_o1)
    out_copy_wait(last - np.int32(1), val0_v, sem_o0)
    out_copy_wait(last, val1_v, sem_o1)


def kernel(indices, hashed_weight):
    idx32 = indices.reshape(-1).astype(jnp.int32)
    w = hashed_weight.astype(jnp.float32)
    out = _emb_kernel(idx32, w, jnp.asarray(_T1), jnp.asarray(_T2))
    return out.reshape(_N, _D)
